# baseline (speedup 1.0000x reference)
import jax
import jax.numpy as jnp
from jax.experimental import pallas as pl


def _add_pe_kernel(x_ref, pe_ref, o_ref):
    i = pl.program_id(0)
    rows = x_ref.shape[0]
    blocks_per_batch = pe_ref.shape[0] // rows
    j = jax.lax.rem(i, blocks_per_batch)
    o_ref[...] = x_ref[...] + pe_ref[pl.ds(j * rows, rows), :]


def kernel(x, pe_weight):
    B, S, D = x.shape
    BS = 1024
    x2 = x.reshape(B * S, D)
    grid = (B * S // BS,)
    out = pl.pallas_call(
        _add_pe_kernel,
        grid=grid,
        in_specs=[
            pl.BlockSpec((BS, D), lambda i: (i, 0)),
            pl.BlockSpec((S, D), lambda i: (0, 0)),
        ],
        out_specs=pl.BlockSpec((BS, D), lambda i: (i, 0)),
        out_shape=jax.ShapeDtypeStruct((B * S, D), x.dtype),
    )(x2, pe_weight)
    return out.reshape(B, S, D)


# speedup vs baseline: 1.0412x; 1.0412x over previous
import jax
import jax.numpy as jnp
from jax.experimental import pallas as pl


def _add_pe_kernel(x_ref, pe_ref, o_ref):
    i = pl.program_id(0)
    rows = x_ref.shape[0]
    blocks_per_batch = pe_ref.shape[0] // rows
    j = jax.lax.rem(i, blocks_per_batch)
    o_ref[...] = x_ref[...] + pe_ref[pl.ds(j * rows, rows), :]


def kernel(x, pe_weight):
    B, S, D = x.shape
    BS = 2048
    x2 = x.reshape(B * S, D)
    grid = (B * S // BS,)
    out = pl.pallas_call(
        _add_pe_kernel,
        grid=grid,
        in_specs=[
            pl.BlockSpec((BS, D), lambda i: (i, 0)),
            pl.BlockSpec((S, D), lambda i: (0, 0)),
        ],
        out_specs=pl.BlockSpec((BS, D), lambda i: (i, 0)),
        out_shape=jax.ShapeDtypeStruct((B * S, D), x.dtype),
    )(x2, pe_weight)
    return out.reshape(B, S, D)
